# Initial kernel scaffold; baseline (speedup 1.0000x reference)
import jax
import jax.numpy as jnp
from jax.experimental import pallas as pl


def kernel(x, edge_index, W1, b1, W2, b2):
    raise NotImplementedError("write your pallas kernel here")


# speedup vs baseline: 7.9974x; 7.9974x over previous
import functools

import jax
import jax.numpy as jnp
from jax import lax
from jax.experimental import pallas as pl
from jax.experimental.pallas import tpu as pltpu
from jax.experimental.pallas import tpu_sc as plsc

N = 10000
E = 160000
IN_CH, HID_CH, OUT_CH = 256, 512, 256

NC, NS, L = 2, 16, 16
CH = 128
NPAD = 10240
B = 128
NB = 84
NBUF = 1
ET = NB * B
EEPAD = NS * ET
EW = EEPAD // (NC * NS)
RPT = NPAD // NS
ZR = 32

_mesh = lambda: plsc.VectorSubcoreMesh(core_axis_name="c", subcore_axis_name="s")
_SC_PARAMS = pltpu.CompilerParams(needs_layout_passes=False)


@functools.partial(
    pl.kernel,
    out_type=jax.ShapeDtypeStruct((NC * NS, NPAD), jnp.float32),
    mesh=_mesh(),
    compiler_params=_SC_PARAMS,
    scratch_types=[
        pltpu.VMEM((EW,), jnp.int32),
        pltpu.VMEM((NPAD,), jnp.float32),
    ],
)
def _sc_hist(dst_hbm, out_hbm, dst_v, hist_v):
    c = lax.axis_index("c")
    s = lax.axis_index("s")
    w = s * NC + c
    pltpu.sync_copy(dst_hbm.at[pl.ds(w * EW, EW)], dst_v)

    zeros16 = jnp.zeros((L,), jnp.float32)

    @pl.loop(0, NPAD // L)
    def _zero(i):
        hist_v[pl.ds(i * L, L)] = zeros16

    ones16 = jnp.ones((L,), jnp.float32)

    @pl.loop(0, EW // L)
    def _acc(i):
        idx = dst_v[pl.ds(i * L, L)]
        plsc.addupdate_scatter(hist_v, [idx], ones16)

    pltpu.sync_copy(hist_v, out_hbm.at[w])


def _make_sc_agg(nc):
    passes = nc // NC

    @functools.partial(
        pl.kernel,
        out_type=jax.ShapeDtypeStruct((nc * NPAD, CH), jnp.float32),
        mesh=_mesh(),
        compiler_params=_SC_PARAMS,
        scratch_types=[
            pltpu.VMEM((NB, B), jnp.int32),
            pltpu.VMEM((NB, B), jnp.int32),
            pltpu.VMEM((NBUF, B, CH), jnp.float32),
            pltpu.VMEM((ZR, CH), jnp.float32),
            pltpu.VMEM_SHARED((NPAD, CH), jnp.float32),
        ]
        + [pltpu.SemaphoreType.DMA] * NBUF,
    )
    def agg(xs_hbm, src_hbm, dst_hbm, out_hbm,
            src_v, dst_v, rowbuf, zbuf, acc, *sems):
        c = lax.axis_index("c")
        s = lax.axis_index("s")
        pltpu.sync_copy(src_hbm.at[s], src_v)
        pltpu.sync_copy(dst_hbm.at[s], dst_v)

        zeros16 = jnp.zeros((L,), jnp.float32)
        cpr = CH // L

        @pl.loop(0, ZR * cpr)
        def _zero(k):
            zbuf[k // cpr, pl.ds((k % cpr) * L, L)] = zeros16

        off0 = c * NPAD

        @pl.loop(0, NB * (B // L))
        def _adj0(k):
            row = k // (B // L)
            col = (k % (B // L)) * L
            src_v[row, pl.ds(col, L)] = src_v[row, pl.ds(col, L)] + off0

        for p in range(passes):
            if p > 0:
                @pl.loop(0, NB * (B // L))
                def _adj(k):
                    row = k // (B // L)
                    col = (k % (B // L)) * L
                    src_v[row, pl.ds(col, L)] = (
                        src_v[row, pl.ds(col, L)] + NC * NPAD)

            for z in range(RPT // ZR):
                pltpu.sync_copy(zbuf, acc.at[pl.ds(s * RPT + z * ZR, ZR)])
            plsc.subcore_barrier()

            for b in range(NBUF):
                pltpu.make_async_copy(
                    xs_hbm.at[src_v.at[b]], rowbuf.at[b], sems[b]).start()

            @pl.loop(0, NB // NBUF - 1)
            def _main(g):
                for b in range(NBUF):
                    j = g * NBUF + b
                    pltpu.make_async_copy(
                        xs_hbm.at[src_v.at[j]], rowbuf.at[b], sems[b]).wait()
                    pltpu.sync_copy(rowbuf.at[b], acc.at[dst_v.at[j]], add=True)
                    pltpu.make_async_copy(
                        xs_hbm.at[src_v.at[j + NBUF]], rowbuf.at[b],
                        sems[b]).start()

            for b in range(NBUF):
                j = NB - NBUF + b
                pltpu.make_async_copy(
                    xs_hbm.at[src_v.at[j]], rowbuf.at[b], sems[b]).wait()
                pltpu.sync_copy(rowbuf.at[b], acc.at[dst_v.at[j]], add=True)

            plsc.subcore_barrier()
            pltpu.sync_copy(
                acc.at[pl.ds(s * RPT, RPT)],
                out_hbm.at[pl.ds((p * NC + c) * NPAD + s * RPT, RPT)])
            if p + 1 < passes:
                plsc.subcore_barrier()

    return agg


_sc_agg1 = _make_sc_agg(HID_CH // CH)
_sc_agg2 = _make_sc_agg(OUT_CH // CH)


_BN = 512


def _tc1_body(deg_ref, x_ref, w_ref, dinv_ref, xs_ref):
    deg = jnp.sum(deg_ref[...], axis=0)
    dinv = jnp.where(deg > 0, lax.rsqrt(deg), 0.0)
    dinv_ref[0] = dinv
    xw = jnp.dot(x_ref[...], w_ref[...], preferred_element_type=jnp.float32)
    for cidx in range(HID_CH // CH):
        xs_ref[cidx] = xw[:, cidx * CH:(cidx + 1) * CH] * dinv[:, None]


def _tc2_body(agg_ref, dinv_ref, b1_ref, w_ref, xs_ref):
    dinv = dinv_ref[0]
    a = jnp.concatenate([agg_ref[cidx] for cidx in range(HID_CH // CH)], axis=1)
    h = jnp.maximum(a * dinv[:, None] + b1_ref[...], 0.0)
    xw = jnp.dot(h, w_ref[...], preferred_element_type=jnp.float32)
    for cidx in range(OUT_CH // CH):
        xs_ref[cidx] = xw[:, cidx * CH:(cidx + 1) * CH] * dinv[:, None]


def _tc3_body(agg_ref, dinv_ref, b2_ref, out_ref):
    dinv = dinv_ref[0]
    a = jnp.concatenate([agg_ref[cidx] for cidx in range(OUT_CH // CH)], axis=1)
    out_ref[...] = a * dinv[:, None] + b2_ref[...]


def _tc1(deg_parts, xpad, W1):
    grid = (NPAD // _BN,)
    return pl.pallas_call(
        _tc1_body,
        grid=grid,
        in_specs=[
            pl.BlockSpec((NC * NS, _BN), lambda i: (0, i)),
            pl.BlockSpec((_BN, IN_CH), lambda i: (i, 0)),
            pl.BlockSpec((IN_CH, HID_CH), lambda i: (0, 0)),
        ],
        out_specs=[
            pl.BlockSpec((1, _BN), lambda i: (0, i)),
            pl.BlockSpec((HID_CH // CH, _BN, CH), lambda i: (0, i, 0)),
        ],
        out_shape=[
            jax.ShapeDtypeStruct((1, NPAD), jnp.float32),
            jax.ShapeDtypeStruct((HID_CH // CH, NPAD, CH), jnp.float32),
        ],
    )(deg_parts, xpad, W1)


def _tc2(agg1, dinv, b1, W2):
    grid = (NPAD // _BN,)
    return pl.pallas_call(
        _tc2_body,
        grid=grid,
        in_specs=[
            pl.BlockSpec((HID_CH // CH, _BN, CH), lambda i: (0, i, 0)),
            pl.BlockSpec((1, _BN), lambda i: (0, i)),
            pl.BlockSpec((1, HID_CH), lambda i: (0, 0)),
            pl.BlockSpec((HID_CH, OUT_CH), lambda i: (0, 0)),
        ],
        out_specs=pl.BlockSpec((OUT_CH // CH, _BN, CH), lambda i: (0, i, 0)),
        out_shape=jax.ShapeDtypeStruct((OUT_CH // CH, NPAD, CH), jnp.float32),
    )(agg1, dinv, b1, W2)


def _tc3(agg2, dinv, b2):
    grid = (NPAD // _BN,)
    return pl.pallas_call(
        _tc3_body,
        grid=grid,
        in_specs=[
            pl.BlockSpec((OUT_CH // CH, _BN, CH), lambda i: (0, i, 0)),
            pl.BlockSpec((1, _BN), lambda i: (0, i)),
            pl.BlockSpec((1, OUT_CH), lambda i: (0, 0)),
        ],
        out_specs=pl.BlockSpec((_BN, OUT_CH), lambda i: (i, 0)),
        out_shape=jax.ShapeDtypeStruct((NPAD, OUT_CH), jnp.float32),
    )(agg2, dinv, b2)


def kernel(x, edge_index, W1, b1, W2, b2):
    n = x.shape[0]
    e = edge_index.shape[1]
    pad = EEPAD - (e + n)

    loop = jnp.arange(n, dtype=jnp.int32)
    padv = jnp.full((pad,), n, dtype=jnp.int32)
    src_ext = jnp.concatenate([edge_index[0], loop, padv])
    dst_ext = jnp.concatenate([edge_index[1], loop, padv])
    src3 = src_ext.reshape(NS, NB, B)
    dst3 = dst_ext.reshape(NS, NB, B)

    xpad = jnp.pad(x, ((0, NPAD - n), (0, 0)))

    deg_parts = _sc_hist(dst_ext)
    dinv, xs1 = _tc1(deg_parts, xpad, W1)
    agg1 = _sc_agg1(xs1.reshape((HID_CH // CH) * NPAD, CH), src3, dst3)
    xs2 = _tc2(agg1.reshape(HID_CH // CH, NPAD, CH), dinv,
               b1.reshape(1, HID_CH), W2)
    agg2 = _sc_agg2(xs2.reshape((OUT_CH // CH) * NPAD, CH), src3, dst3)
    out = _tc3(agg2.reshape(OUT_CH // CH, NPAD, CH), dinv,
               b2.reshape(1, OUT_CH))
    return out[:n]
